# Initial kernel scaffold; baseline (speedup 1.0000x reference)
import jax
import jax.numpy as jnp
from jax.experimental import pallas as pl


def kernel(features, weight_values, bias, conn_out, conn_in):
    raise NotImplementedError("write your pallas kernel here")


# speedup vs baseline: 4.4788x; 4.4788x over previous
import functools

import jax
import jax.numpy as jnp
from jax import lax
from jax.experimental import pallas as pl
from jax.experimental.pallas import tpu as pltpu
from jax.experimental.pallas import tpu_sc as plsc

IN_F = 4096
OUT_F = 4096
NNZ = 167772
BATCH = 1024
NEG_SLOPE = 0.01

NUM_CORES = 2
NUM_SUBCORES = 16

BAND_ROWS = 128
NUM_BANDS = OUT_F // BAND_ROWS
BANDS_PER_CORE = NUM_BANDS // NUM_CORES
BAND_ELEMS = BAND_ROWS * IN_F
ZONE = BAND_ELEMS // NUM_SUBCORES

CHUNK = 128
SLICE = 10496
NNZ_PAD = SLICE * NUM_SUBCORES
NCHUNKS = SLICE // CHUNK
DEPTH = 8

ZBUF = 16384
NZDMA = ZONE // ZBUF


def _sc_body(co_hbm, ci_hbm, wv_hbm, w_hbm,
             co_v, ci_v, wv_v, idx_c, val_c, zbuf, shared, sem):
    c = lax.axis_index("c")
    s = lax.axis_index("s")
    base = s * SLICE
    pltpu.sync_copy(co_hbm.at[pl.ds(base, SLICE)], co_v)
    pltpu.sync_copy(ci_hbm.at[pl.ds(base, SLICE)], ci_v)
    pltpu.sync_copy(wv_hbm.at[pl.ds(base, SLICE)], wv_v)

    def pre(j, carry):
        o = co_v[pl.ds(j * 16, 16)]
        i = ci_v[pl.ds(j * 16, 16)]
        co_v[pl.ds(j * 16, 16)] = o >> 7
        ci_v[pl.ds(j * 16, 16)] = (o & 127) * IN_F + i
        return carry
    lax.fori_loop(0, SLICE // 16, pre, 0)

    zeros16 = jnp.zeros((16,), jnp.float32)

    def zf(j, carry):
        zbuf[pl.ds(j * 16, 16)] = zeros16
        return carry
    lax.fori_loop(0, ZBUF // 16, zf, 0)

    pad_idx = lax.iota(jnp.int32, 16) * 16 + s * 256
    zone_base = s * ZONE

    def band_step(p, carry):
        band = c * BANDS_PER_CORE + p
        for k in range(NZDMA):
            pltpu.async_copy(
                zbuf, shared.at[pl.ds(zone_base + k * ZBUF, ZBUF)], sem)
        for k in range(NZDMA):
            pltpu.make_async_copy(
                zbuf, shared.at[pl.ds(zone_base + k * ZBUF, ZBUF)], sem
            ).wait()
        plsc.subcore_barrier()

        def chunk_step(t, carry):
            for u in range(CHUNK // 16):
                j = t * CHUNK + u * 16
                b = co_v[pl.ds(j, 16)]
                l = ci_v[pl.ds(j, 16)]
                v = wv_v[pl.ds(j, 16)]
                m = b == band
                idx_c[t, pl.ds(u * 16, 16)] = jnp.where(m, l, pad_idx)
                val_c[t, pl.ds(u * 16, 16)] = jnp.where(m, v, 0.0)
            pltpu.async_copy(val_c.at[t], shared.at[idx_c.at[t]], sem,
                             add=True)

            @pl.when(t >= DEPTH)
            def _():
                pltpu.make_async_copy(
                    val_c.at[0], shared.at[idx_c.at[0]], sem).wait()
            return carry
        lax.fori_loop(0, NCHUNKS, chunk_step, 0)

        def drain(t, carry):
            pltpu.make_async_copy(
                val_c.at[0], shared.at[idx_c.at[0]], sem).wait()
            return carry
        lax.fori_loop(0, DEPTH, drain, 0)
        plsc.subcore_barrier()

        pltpu.sync_copy(
            shared.at[pl.ds(zone_base, ZONE)],
            w_hbm.at[pl.ds(band * BAND_ELEMS + zone_base, ZONE)])
        return carry
    lax.fori_loop(0, BANDS_PER_CORE, band_step, 0)


def _build_w(co, ci, wv):
    mesh = plsc.VectorSubcoreMesh(core_axis_name="c", subcore_axis_name="s")
    return pl.kernel(
        _sc_body,
        out_type=jax.ShapeDtypeStruct((OUT_F * IN_F,), jnp.float32),
        mesh=mesh,
        scratch_types=[
            pltpu.VMEM((SLICE,), jnp.int32),
            pltpu.VMEM((SLICE,), jnp.int32),
            pltpu.VMEM((SLICE,), jnp.float32),
            pltpu.VMEM((NCHUNKS, CHUNK), jnp.int32),
            pltpu.VMEM((NCHUNKS, CHUNK), jnp.float32),
            pltpu.VMEM((ZBUF,), jnp.float32),
            pltpu.VMEM_SHARED((BAND_ELEMS,), jnp.float32),
            pltpu.SemaphoreType.DMA,
        ],
    )(co, ci, wv)


BN = 512
N_BLK = OUT_F // BN


def _mm_body(x_ref, w_ref, b_ref, o_ref):
    x = x_ref[...]
    w = w_ref[...]
    y = lax.dot_general(x, w, (((1,), (1,)), ((), ())),
                        preferred_element_type=jnp.float32)
    y = y + b_ref[0]
    o_ref[...] = jnp.where(y >= 0, y, jnp.float32(NEG_SLOPE) * y)


def _matmul(x, w, b3):
    return pl.pallas_call(
        _mm_body,
        grid=(N_BLK,),
        in_specs=[
            pl.BlockSpec((BATCH, IN_F), lambda n: (0, 0)),
            pl.BlockSpec((BN, IN_F), lambda n: (n, 0)),
            pl.BlockSpec((1, 1, BN), lambda n: (n, 0, 0)),
        ],
        out_specs=pl.BlockSpec((BATCH, BN), lambda n: (0, n)),
        out_shape=jax.ShapeDtypeStruct((BATCH, OUT_F), jnp.float32),
    )(x, w, b3)


def kernel(features, weight_values, bias, conn_out, conn_in):
    pad = NNZ_PAD - NNZ
    co = jnp.pad(conn_out, (0, pad))
    ci = jnp.pad(conn_in, (0, pad))
    wv = jnp.pad(weight_values, (0, pad))
    w_flat = _build_w(co, ci, wv)
    w = w_flat.reshape(OUT_F, IN_F)
    return _matmul(features, w, bias.reshape(N_BLK, 1, BN))


# speedup vs baseline: 5.4391x; 1.2144x over previous
import functools

import jax
import jax.numpy as jnp
from jax import lax
from jax.experimental import pallas as pl
from jax.experimental.pallas import tpu as pltpu
from jax.experimental.pallas import tpu_sc as plsc

IN_F = 4096
OUT_F = 4096
NNZ = 167772
BATCH = 1024
NEG_SLOPE = 0.01

NUM_CORES = 2
NUM_SUBCORES = 16

BAND_ROWS = 256
NUM_BANDS = OUT_F // BAND_ROWS
BANDS_PER_CORE = NUM_BANDS // NUM_CORES
BAND_ELEMS = BAND_ROWS * IN_F
BAND_SHIFT = 20
BAND_MASK = BAND_ELEMS - 1
ZONE = BAND_ELEMS // NUM_SUBCORES

CHUNK = 128
SLICE = 10496
NNZ_PAD = SLICE * NUM_SUBCORES
NCHUNKS = SLICE // CHUNK
DEPTH = 8

ZBUF = 8192
NZDMA = ZONE // ZBUF


def _sc_body(co_hbm, ci_hbm, wv_hbm, w_hbm,
             g_v, wv_v, idx_c, val_c, zbuf, shared, sem):
    c = lax.axis_index("c")
    s = lax.axis_index("s")
    base = s * SLICE
    pltpu.sync_copy(ci_hbm.at[pl.ds(base, SLICE)], g_v)
    pltpu.sync_copy(wv_hbm.at[pl.ds(base, SLICE)], wv_v)

    def ld(t, carry):
        pltpu.async_copy(co_hbm.at[pl.ds(base + t * CHUNK, CHUNK)],
                         idx_c.at[t], sem)
        return carry
    lax.fori_loop(0, NCHUNKS, ld, 0)

    def ld_drain(t, carry):
        pltpu.make_async_copy(co_hbm.at[pl.ds(base, CHUNK)],
                              idx_c.at[0], sem).wait()
        return carry
    lax.fori_loop(0, NCHUNKS, ld_drain, 0)

    def pre(j, carry):
        o = idx_c[j // 8, pl.ds((j % 8) * 16, 16)]
        i = g_v[pl.ds(j * 16, 16)]
        g_v[pl.ds(j * 16, 16)] = o * IN_F + i
        return carry
    lax.fori_loop(0, SLICE // 16, pre, 0)

    zeros16 = jnp.zeros((16,), jnp.float32)

    def zf(j, carry):
        zbuf[pl.ds(j * 16, 16)] = zeros16
        return carry
    lax.fori_loop(0, ZBUF // 16, zf, 0)

    pad_idx = lax.iota(jnp.int32, 16) * 16 + s * 256
    zone_base = s * ZONE

    def band_step(p, carry):
        band = c * BANDS_PER_CORE + p
        band_base = band * BAND_ELEMS
        for k in range(NZDMA):
            pltpu.async_copy(
                zbuf, shared.at[pl.ds(zone_base + k * ZBUF, ZBUF)], sem)
        for k in range(NZDMA):
            pltpu.make_async_copy(
                zbuf, shared.at[pl.ds(zone_base + k * ZBUF, ZBUF)], sem
            ).wait()
        plsc.subcore_barrier()

        def chunk_step(t, carry):
            for u in range(CHUNK // 16):
                j = t * CHUNK + u * 16
                g = g_v[pl.ds(j, 16)]
                v = wv_v[pl.ds(j, 16)]
                m = (g >> BAND_SHIFT) == band
                idx_c[t, pl.ds(u * 16, 16)] = jnp.where(m, g & BAND_MASK,
                                                        pad_idx)
                val_c[t, pl.ds(u * 16, 16)] = jnp.where(m, v, 0.0)
            pltpu.async_copy(val_c.at[t], shared.at[idx_c.at[t]], sem,
                             add=True)

            @pl.when(t >= DEPTH)
            def _():
                pltpu.make_async_copy(
                    val_c.at[0], shared.at[idx_c.at[0]], sem).wait()
            return carry
        lax.fori_loop(0, NCHUNKS, chunk_step, 0)

        def drain(t, carry):
            pltpu.make_async_copy(
                val_c.at[0], shared.at[idx_c.at[0]], sem).wait()
            return carry
        lax.fori_loop(0, DEPTH, drain, 0)
        plsc.subcore_barrier()

        pltpu.sync_copy(
            shared.at[pl.ds(zone_base, ZONE)],
            w_hbm.at[pl.ds(band_base + zone_base, ZONE)])
        return carry
    lax.fori_loop(0, BANDS_PER_CORE, band_step, 0)


def _build_w(co, ci, wv):
    mesh = plsc.VectorSubcoreMesh(core_axis_name="c", subcore_axis_name="s")
    return pl.kernel(
        _sc_body,
        out_type=jax.ShapeDtypeStruct((OUT_F * IN_F,), jnp.float32),
        mesh=mesh,
        scratch_types=[
            pltpu.VMEM((SLICE,), jnp.int32),
            pltpu.VMEM((SLICE,), jnp.float32),
            pltpu.VMEM((NCHUNKS, CHUNK), jnp.int32),
            pltpu.VMEM((NCHUNKS, CHUNK), jnp.float32),
            pltpu.VMEM((ZBUF,), jnp.float32),
            pltpu.VMEM_SHARED((BAND_ELEMS,), jnp.float32),
            pltpu.SemaphoreType.DMA,
        ],
    )(co, ci, wv)


BN = 512
N_BLK = OUT_F // BN


def _mm_body(x_ref, w_ref, b_ref, o_ref):
    x = x_ref[...]
    w = w_ref[...].astype(jnp.bfloat16)
    y = lax.dot_general(x, w, (((1,), (1,)), ((), ())),
                        preferred_element_type=jnp.float32)
    y = y + b_ref[0]
    o_ref[...] = jnp.where(y >= 0, y, jnp.float32(NEG_SLOPE) * y)


def _matmul(x, w, b3):
    return pl.pallas_call(
        _mm_body,
        grid=(N_BLK,),
        in_specs=[
            pl.BlockSpec((BATCH, IN_F), lambda n: (0, 0)),
            pl.BlockSpec((BN, IN_F), lambda n: (n, 0)),
            pl.BlockSpec((1, 1, BN), lambda n: (n, 0, 0)),
        ],
        out_specs=pl.BlockSpec((BATCH, BN), lambda n: (0, n)),
        out_shape=jax.ShapeDtypeStruct((BATCH, OUT_F), jnp.float32),
    )(x, w, b3)


def kernel(features, weight_values, bias, conn_out, conn_in):
    pad = NNZ_PAD - NNZ
    co = jnp.pad(conn_out, (0, pad))
    ci = jnp.pad(conn_in, (0, pad))
    wv = jnp.pad(weight_values, (0, pad))
    w_flat = _build_w(co, ci, wv)
    w = w_flat.reshape(OUT_F, IN_F)
    x16 = features.astype(jnp.bfloat16)
    return _matmul(x16, w, bias.reshape(N_BLK, 1, BN))


# speedup vs baseline: 6.1775x; 1.1357x over previous
import functools

import jax
import jax.numpy as jnp
from jax import lax
from jax.experimental import pallas as pl
from jax.experimental.pallas import tpu as pltpu
from jax.experimental.pallas import tpu_sc as plsc

IN_F = 4096
OUT_F = 4096
NNZ = 167772
BATCH = 1024
NEG_SLOPE = 0.01

NUM_CORES = 2
NUM_SUBCORES = 16

BAND_ROWS = 256
NUM_BANDS = OUT_F // BAND_ROWS
BANDS_PER_CORE = NUM_BANDS // NUM_CORES
BAND_ELEMS = BAND_ROWS * IN_F
BAND_SHIFT = 20
BAND_MASK = BAND_ELEMS - 1
ZONE = BAND_ELEMS // NUM_SUBCORES
ZROWS = BAND_ROWS // NUM_SUBCORES

CHUNK = 128
SLICE = 10496
NNZ_PAD = SLICE * NUM_SUBCORES
NCHUNKS = SLICE // CHUNK
RING = 8

ZBUF = 8192
NZDMA = ZONE // ZBUF

_MESH = plsc.VectorSubcoreMesh(core_axis_name="c", subcore_axis_name="s")


def _sc_body(co_hbm, ci_hbm, wv_hbm, w_hbm,
             g_v, wv_v, idx_r, val_r, zbuf, shared, sem):
    c = lax.axis_index("c")
    s = lax.axis_index("s")
    base = s * SLICE
    pltpu.sync_copy(ci_hbm.at[pl.ds(base, SLICE)], g_v)
    pltpu.sync_copy(wv_hbm.at[pl.ds(base, SLICE)], wv_v)

    def ld(t, carry):
        pltpu.sync_copy(co_hbm.at[pl.ds(base + t * CHUNK, CHUNK)],
                        idx_r.at[t % RING])

        def pre(u, carry2):
            o = idx_r[t % RING, pl.ds(u * 16, 16)]
            j = t * CHUNK + u * 16
            i = g_v[pl.ds(j, 16)]
            g_v[pl.ds(j, 16)] = o * IN_F + i
            return carry2
        lax.fori_loop(0, CHUNK // 16, pre, 0)
        return carry
    lax.fori_loop(0, NCHUNKS, ld, 0)

    zeros16 = jnp.zeros((16,), jnp.float32)

    def zf(j, carry):
        zbuf[pl.ds(j * 16, 16)] = zeros16
        return carry
    lax.fori_loop(0, ZBUF // 16, zf, 0)

    lanes = lax.iota(jnp.int32, 16)
    pad_idx = lanes * 16 + s * 256
    zone_base = s * ZONE

    def band_step(p, carry):
        band = c * BANDS_PER_CORE + p
        for k in range(NZDMA):
            pltpu.async_copy(
                zbuf, shared.at[pl.ds(zone_base + k * ZBUF, ZBUF)], sem)
        for k in range(NZDMA):
            pltpu.make_async_copy(
                zbuf, shared.at[pl.ds(zone_base + k * ZBUF, ZBUF)], sem
            ).wait()
        plsc.subcore_barrier()

        def chunk_step(t, carry):
            @pl.when(t >= RING)
            def _():
                pltpu.make_async_copy(val_r.at[0],
                                      shared.at[idx_r.at[0]], sem).wait()
            r = t % RING
            for u in range(CHUNK // 16):
                j = t * CHUNK + u * 16
                g = g_v[pl.ds(j, 16)]
                v = wv_v[pl.ds(j, 16)]
                m = (g >> BAND_SHIFT) == band
                idx_r[r, pl.ds(u * 16, 16)] = jnp.where(m, g & BAND_MASK,
                                                        pad_idx)
                val_r[r, pl.ds(u * 16, 16)] = jnp.where(m, v, 0.0)
            pltpu.async_copy(val_r.at[r], shared.at[idx_r.at[r]], sem,
                             add=True)
            return carry
        lax.fori_loop(0, NCHUNKS, chunk_step, 0)

        def drain(t, carry):
            pltpu.make_async_copy(val_r.at[0],
                                  shared.at[idx_r.at[0]], sem).wait()
            return carry
        lax.fori_loop(0, RING, drain, 0)
        plsc.subcore_barrier()

        row0 = band * BAND_ROWS + s * ZROWS

        def crow(r, carry):
            pltpu.async_copy(
                shared.at[pl.ds(zone_base + r * IN_F, IN_F)],
                w_hbm.at[row0 + r], sem)
            return carry
        lax.fori_loop(0, ZROWS, crow, 0)

        def crow_drain(r, carry):
            pltpu.make_async_copy(
                shared.at[pl.ds(zone_base, IN_F)], w_hbm.at[0], sem).wait()
            return carry
        lax.fori_loop(0, ZROWS, crow_drain, 0)
        return carry
    lax.fori_loop(0, BANDS_PER_CORE, band_step, 0)


def _build_w(co, ci, wv):
    return pl.kernel(
        _sc_body,
        out_type=jax.ShapeDtypeStruct((OUT_F, IN_F), jnp.float32),
        mesh=_MESH,
        scratch_types=[
            pltpu.VMEM((SLICE,), jnp.int32),
            pltpu.VMEM((SLICE,), jnp.float32),
            pltpu.VMEM((RING, CHUNK), jnp.int32),
            pltpu.VMEM((RING, CHUNK), jnp.float32),
            pltpu.VMEM((ZBUF,), jnp.float32),
            pltpu.VMEM_SHARED((BAND_ELEMS,), jnp.float32),
            pltpu.SemaphoreType.DMA,
        ],
    )(co, ci, wv)


BN = 512
N_BLK = OUT_F // BN


def _mm_body(x_ref, w_ref, b_ref, o_ref):
    x = x_ref[...].astype(jnp.bfloat16)
    w = w_ref[...].astype(jnp.bfloat16)
    y = lax.dot_general(x, w, (((1,), (1,)), ((), ())),
                        preferred_element_type=jnp.float32)
    y = y + b_ref[0]
    o_ref[...] = jnp.where(y >= 0, y, jnp.float32(NEG_SLOPE) * y)


def _matmul(x, w, b3):
    return pl.pallas_call(
        _mm_body,
        grid=(N_BLK,),
        in_specs=[
            pl.BlockSpec((BATCH, IN_F), lambda n: (0, 0)),
            pl.BlockSpec((BN, IN_F), lambda n: (n, 0)),
            pl.BlockSpec((1, 1, BN), lambda n: (n, 0, 0)),
        ],
        out_specs=pl.BlockSpec((BATCH, BN), lambda n: (0, n)),
        out_shape=jax.ShapeDtypeStruct((BATCH, OUT_F), jnp.float32),
    )(x, w, b3)


def kernel(features, weight_values, bias, conn_out, conn_in):
    pad = NNZ_PAD - NNZ
    co = jnp.pad(conn_out, (0, pad))
    ci = jnp.pad(conn_in, (0, pad))
    wv = jnp.pad(weight_values, (0, pad))
    w = _build_w(co, ci, wv)
    y = _matmul(features, w, bias.reshape(N_BLK, 1, BN))
    return y


# speedup vs baseline: 7.3754x; 1.1939x over previous
import functools

import jax
import jax.numpy as jnp
from jax import lax
from jax.experimental import pallas as pl
from jax.experimental.pallas import tpu as pltpu
from jax.experimental.pallas import tpu_sc as plsc

IN_F = 4096
OUT_F = 4096
NNZ = 167772
BATCH = 1024
NEG_SLOPE = 0.01

NUM_CORES = 2
NUM_SUBCORES = 16

BAND_ROWS = 256
NUM_BANDS = OUT_F // BAND_ROWS
BANDS_PER_CORE = NUM_BANDS // NUM_CORES
BAND_ELEMS = BAND_ROWS * IN_F
BAND_SHIFT = 20
BAND_MASK = BAND_ELEMS - 1
ZONE = BAND_ELEMS // NUM_SUBCORES
ZROWS = BAND_ROWS // NUM_SUBCORES

CHUNK = 128
SLICE = 10496
NNZ_PAD = SLICE * NUM_SUBCORES
NCHUNKS = SLICE // CHUNK
DEPTH = 8

ZBUF = 8192
NZDMA = ZONE // ZBUF

_MESH = plsc.VectorSubcoreMesh(core_axis_name="c", subcore_axis_name="s")


def _sc_body(co_hbm, ci_hbm, wv_hbm, w_hbm,
             g_v, wv_v, idx_c, val_c, zbuf, shared, sem):
    c = lax.axis_index("c")
    s = lax.axis_index("s")
    base = s * SLICE
    pltpu.sync_copy(ci_hbm.at[pl.ds(base, SLICE)], g_v)
    pltpu.sync_copy(wv_hbm.at[pl.ds(base, SLICE)], wv_v)

    def ld(t, carry):
        pltpu.async_copy(co_hbm.at[pl.ds(base + t * CHUNK, CHUNK)],
                         idx_c.at[t], sem)
        return carry
    lax.fori_loop(0, NCHUNKS, ld, 0)

    def ld_drain(t, carry):
        pltpu.make_async_copy(co_hbm.at[pl.ds(base, CHUNK)],
                              idx_c.at[0], sem).wait()
        return carry
    lax.fori_loop(0, NCHUNKS, ld_drain, 0)

    def pre(j, carry):
        o = idx_c[j // 8, pl.ds((j % 8) * 16, 16)]
        i = g_v[pl.ds(j * 16, 16)]
        g_v[pl.ds(j * 16, 16)] = o * IN_F + i
        return carry
    lax.fori_loop(0, SLICE // 16, pre, 0)

    zeros16 = jnp.zeros((16,), jnp.float32)

    def zf(j, carry):
        zbuf[pl.ds(j * 16, 16)] = zeros16
        return carry
    lax.fori_loop(0, ZBUF // 16, zf, 0)

    pad_idx = lax.iota(jnp.int32, 16) * 16 + s * 256
    zone_base = s * ZONE

    def band_step(p, carry):
        band = c * BANDS_PER_CORE + p
        for k in range(NZDMA):
            pltpu.async_copy(
                zbuf, shared.at[pl.ds(zone_base + k * ZBUF, ZBUF)], sem)

        def sel(t, carry2):
            for u in range(CHUNK // 16):
                j = t * CHUNK + u * 16
                g = g_v[pl.ds(j, 16)]
                v = wv_v[pl.ds(j, 16)]
                m = (g >> BAND_SHIFT) == band
                idx_c[t, pl.ds(u * 16, 16)] = jnp.where(m, g & BAND_MASK,
                                                        pad_idx)
                val_c[t, pl.ds(u * 16, 16)] = jnp.where(m, v, 0.0)
            return carry2
        lax.fori_loop(0, NCHUNKS, sel, 0)

        for k in range(NZDMA):
            pltpu.make_async_copy(
                zbuf, shared.at[pl.ds(zone_base + k * ZBUF, ZBUF)], sem
            ).wait()
        plsc.subcore_barrier()

        def fire(t, carry2):
            pltpu.async_copy(val_c.at[t], shared.at[idx_c.at[t]], sem,
                             add=True)

            @pl.when(t >= DEPTH)
            def _():
                pltpu.make_async_copy(
                    val_c.at[0], shared.at[idx_c.at[0]], sem).wait()
            return carry2
        lax.fori_loop(0, NCHUNKS, fire, 0)

        def drain(t, carry2):
            pltpu.make_async_copy(
                val_c.at[0], shared.at[idx_c.at[0]], sem).wait()
            return carry2
        lax.fori_loop(0, DEPTH, drain, 0)
        plsc.subcore_barrier()

        row0 = band * BAND_ROWS + s * ZROWS

        def crow(r, carry2):
            pltpu.async_copy(
                shared.at[pl.ds(zone_base + r * IN_F, IN_F)],
                w_hbm.at[row0 + r], sem)
            return carry2
        lax.fori_loop(0, ZROWS, crow, 0)

        def crow_drain(r, carry2):
            pltpu.make_async_copy(
                shared.at[pl.ds(zone_base, IN_F)], w_hbm.at[0], sem).wait()
            return carry2
        lax.fori_loop(0, ZROWS, crow_drain, 0)
        return carry
    lax.fori_loop(0, BANDS_PER_CORE, band_step, 0)


def _build_w(co, ci, wv):
    return pl.kernel(
        _sc_body,
        out_type=jax.ShapeDtypeStruct((OUT_F, IN_F), jnp.float32),
        mesh=_MESH,
        scratch_types=[
            pltpu.VMEM((SLICE,), jnp.int32),
            pltpu.VMEM((SLICE,), jnp.float32),
            pltpu.VMEM((NCHUNKS, CHUNK), jnp.int32),
            pltpu.VMEM((NCHUNKS, CHUNK), jnp.float32),
            pltpu.VMEM((ZBUF,), jnp.float32),
            pltpu.VMEM_SHARED((BAND_ELEMS,), jnp.float32),
            pltpu.SemaphoreType.DMA,
        ],
    )(co, ci, wv)


BN = 512
N_BLK = OUT_F // BN


def _mm_body(x_ref, w_ref, b_ref, o_ref):
    x = x_ref[...].astype(jnp.bfloat16)
    w = w_ref[...].astype(jnp.bfloat16)
    y = lax.dot_general(x, w, (((1,), (1,)), ((), ())),
                        preferred_element_type=jnp.float32)
    y = y + b_ref[0]
    o_ref[...] = jnp.where(y >= 0, y, jnp.float32(NEG_SLOPE) * y)


def _matmul(x, w, b3):
    return pl.pallas_call(
        _mm_body,
        grid=(N_BLK,),
        in_specs=[
            pl.BlockSpec((BATCH, IN_F), lambda n: (0, 0)),
            pl.BlockSpec((BN, IN_F), lambda n: (n, 0)),
            pl.BlockSpec((1, 1, BN), lambda n: (n, 0, 0)),
        ],
        out_specs=pl.BlockSpec((BATCH, BN), lambda n: (0, n)),
        out_shape=jax.ShapeDtypeStruct((BATCH, OUT_F), jnp.float32),
    )(x, w, b3)


def kernel(features, weight_values, bias, conn_out, conn_in):
    pad = NNZ_PAD - NNZ
    co = jnp.pad(conn_out, (0, pad))
    ci = jnp.pad(conn_in, (0, pad))
    wv = jnp.pad(weight_values, (0, pad))
    w = _build_w(co, ci, wv)
    return _matmul(features, w, bias.reshape(N_BLK, 1, BN))
